# Initial kernel scaffold; baseline (speedup 1.0000x reference)
import jax
import jax.numpy as jnp
from jax.experimental import pallas as pl


def kernel(dofs, parent):
    raise NotImplementedError("write your pallas kernel here")


# speedup vs baseline: 8.3213x; 8.3213x over previous
import functools

import jax
import jax.numpy as jnp
from jax import lax
from jax.experimental import pallas as pl
from jax.experimental.pallas import tpu as pltpu
from jax.experimental.pallas import tpu_sc as plsc

L = 16
NC, NS = 2, 16
NW = NC * NS
GCHUNK = 128
PD_ROUNDS = 8


def _tc_build_body(dofs_ref, out_ref):
    phi_p = dofs_ref[0]
    theta = dofs_ref[1]
    d = dofs_ref[2]
    phi_c = dofs_ref[3]

    cp, sp = jnp.cos(0.5 * phi_p), jnp.sin(0.5 * phi_p)
    ct, st = jnp.cos(0.5 * theta), jnp.sin(0.5 * theta)
    cc, sc_ = jnp.cos(0.5 * phi_c), jnp.sin(0.5 * phi_c)
    w1 = cp * ct
    x1 = sp * ct
    y1 = -sp * st
    z1 = cp * st
    qw = w1 * cc - x1 * sc_
    qx = w1 * sc_ + x1 * cc
    qy = y1 * cc + z1 * sc_
    qz = z1 * cc - y1 * sc_
    ctf, stf = jnp.cos(theta), jnp.sin(theta)
    cpf, spf = jnp.cos(phi_p), jnp.sin(phi_p)
    tx = d * ctf
    ty = cpf * d * stf
    tz = spf * d * stf

    sub = lax.broadcasted_iota(jnp.int32, (8, 128), 0)
    lane = lax.broadcasted_iota(jnp.int32, (8, 128), 1)
    gidx = (pl.program_id(0) * 8 + sub) * 128 + lane
    is_root = gidx == 0
    zero = jnp.zeros_like(qw)
    one = jnp.ones_like(qw)
    out_ref[0] = jnp.where(is_root, one, qw)
    out_ref[1] = jnp.where(is_root, zero, qx)
    out_ref[2] = jnp.where(is_root, zero, qy)
    out_ref[3] = jnp.where(is_root, zero, qz)
    out_ref[4] = jnp.where(is_root, zero, tx)
    out_ref[5] = jnp.where(is_root, zero, ty)
    out_ref[6] = jnp.where(is_root, zero, tz)


def _tc_build(dofs_planes, npad):
    nrow = npad // 128
    grid = nrow // 8
    return pl.pallas_call(
        _tc_build_body,
        grid=(grid,),
        in_specs=[pl.BlockSpec((4, 8, 128), lambda i: (0, i, 0))],
        out_specs=pl.BlockSpec((7, 8, 128), lambda i: (0, i, 0)),
        out_shape=jax.ShapeDtypeStruct((7, nrow, 128), jnp.float32),
    )(dofs_planes)


def _compose_group(par, own, out, g):
    ii = lax.iota(jnp.int32, L)
    r0 = ii + g * L

    def col(c):
        return lax.broadcast(jnp.int32(c), (L,))

    def loadf(ref, c):
        return plsc.bitcast(plsc.load_gather(ref, [r0, col(c)]), jnp.float32)

    pw = loadf(par, 0)
    px = loadf(par, 1)
    py = loadf(par, 2)
    pz = loadf(par, 3)
    ptx = loadf(par, 4)
    pty = loadf(par, 5)
    ptz = loadf(par, 6)
    pa = plsc.load_gather(par, [r0, col(7)])

    cw = loadf(own, 0)
    cx = loadf(own, 1)
    cy = loadf(own, 2)
    cz = loadf(own, 3)
    ctx = loadf(own, 4)
    cty = loadf(own, 5)
    ctz = loadf(own, 6)

    nw_ = pw * cw - px * cx - py * cy - pz * cz
    nx = pw * cx + px * cw + py * cz - pz * cy
    ny = pw * cy - px * cz + py * cw + pz * cx
    nz = pw * cz + px * cy - py * cx + pz * cw
    ux = py * ctz - pz * cty
    uy = pz * ctx - px * ctz
    uz = px * cty - py * ctx
    wx = py * uz - pz * uy
    wy = pz * ux - px * uz
    wz = px * uy - py * ux
    two = jnp.float32(2.0)
    ntx = ptx + ctx + two * (pw * ux + wx)
    nty = pty + cty + two * (pw * uy + wy)
    ntz = ptz + ctz + two * (pw * uz + wz)

    def storef(c, v):
        plsc.store_scatter(out, [r0, col(c)], plsc.bitcast(v, jnp.int32))

    storef(0, nw_)
    storef(1, nx)
    storef(2, ny)
    storef(3, nz)
    storef(4, ntx)
    storef(5, nty)
    storef(6, ntz)
    plsc.store_scatter(out, [r0, col(7)], pa)


def _pd_step_body(a_hbm, b_hbm, own, par, out, idx, sem, *, wpn):
    wid = lax.axis_index("s") * NC + lax.axis_index("c")
    base = wid * wpn
    pltpu.sync_copy(a_hbm.at[pl.ds(base, wpn)], own)

    def ext_body(g, carry):
        ii = lax.iota(jnp.int32, L)
        r0 = ii + g * L
        c7 = lax.broadcast(jnp.int32(7), (L,))
        idx[pl.ds(g * L, L)] = plsc.load_gather(own, [r0, c7])
        return carry

    lax.fori_loop(0, wpn // L, ext_body, 0)

    nchunk = wpn // GCHUNK
    descs = []
    for c in range(nchunk):
        descs.append(
            pltpu.async_copy(
                a_hbm.at[idx.at[pl.ds(c * GCHUNK, GCHUNK)]],
                par.at[pl.ds(c * GCHUNK, GCHUNK)],
                sem,
            )
        )
    for dsc in descs:
        dsc.wait()

    def comp_body(g, carry):
        _compose_group(par, own, out, g)
        return carry

    lax.fori_loop(0, wpn // L, comp_body, 0)

    pltpu.sync_copy(out, b_hbm.at[pl.ds(base, wpn)])


def _pd_step(a, npad, *, interpret=False):
    wpn = npad // NW
    mesh = plsc.VectorSubcoreMesh(
        core_axis_name="c", subcore_axis_name="s", num_cores=NC, num_subcores=NS
    )
    return pl.kernel(
        functools.partial(_pd_step_body, wpn=wpn),
        out_type=jax.ShapeDtypeStruct((npad, 8), jnp.int32),
        mesh=mesh,
        compiler_params=pltpu.CompilerParams(
            use_tc_tiling_on_sc=False, needs_layout_passes=False
        ),
        scratch_types=[
            pltpu.VMEM((wpn, 8), jnp.int32),
            pltpu.VMEM((wpn, 8), jnp.int32),
            pltpu.VMEM((wpn, 8), jnp.int32),
            pltpu.VMEM((wpn,), jnp.int32),
            pltpu.SemaphoreType.DMA,
        ],
        interpret=interpret,
    )(a)


def kernel(dofs, parent):
    n = dofs.shape[0]
    npad = -(-n // (NW * GCHUNK)) * (NW * GCHUNK)

    dofs4 = jnp.pad(dofs[:, :4], ((0, npad - n), (0, 0)))
    dofs_planes = dofs4.T.reshape(4, npad // 128, 128)
    planes = _tc_build(dofs_planes, npad).reshape(7, npad)

    anc = jnp.pad(parent.astype(jnp.int32), (0, npad - n))
    planes_i = lax.bitcast_convert_type(planes, jnp.int32)
    rows = jnp.concatenate([planes_i, anc[None, :]], axis=0).T

    for _ in range(PD_ROUNDS):
        rows = _pd_step(rows, npad)

    return lax.bitcast_convert_type(rows[:n, 4:7], jnp.float32)


# speedup vs baseline: 35.5476x; 4.2719x over previous
import functools

import jax
import jax.numpy as jnp
from jax import lax
from jax.experimental import pallas as pl
from jax.experimental.pallas import tpu as pltpu
from jax.experimental.pallas import tpu_sc as plsc

L = 16
NC, NS = 2, 16
NW = NC * NS
GCHUNK = 128
PD_ROUNDS = 8


def _tc_build_body(dofs_ref, out_ref):
    phi_p = dofs_ref[0]
    theta = dofs_ref[1]
    d = dofs_ref[2]
    phi_c = dofs_ref[3]

    cp, sp = jnp.cos(0.5 * phi_p), jnp.sin(0.5 * phi_p)
    ct, st = jnp.cos(0.5 * theta), jnp.sin(0.5 * theta)
    cc, sc_ = jnp.cos(0.5 * phi_c), jnp.sin(0.5 * phi_c)
    w1 = cp * ct
    x1 = sp * ct
    y1 = -sp * st
    z1 = cp * st
    qw = w1 * cc - x1 * sc_
    qx = w1 * sc_ + x1 * cc
    qy = y1 * cc + z1 * sc_
    qz = z1 * cc - y1 * sc_
    ctf, stf = jnp.cos(theta), jnp.sin(theta)
    cpf, spf = jnp.cos(phi_p), jnp.sin(phi_p)
    tx = d * ctf
    ty = cpf * d * stf
    tz = spf * d * stf

    sub = lax.broadcasted_iota(jnp.int32, (8, 128), 0)
    lane = lax.broadcasted_iota(jnp.int32, (8, 128), 1)
    gidx = (pl.program_id(0) * 8 + sub) * 128 + lane
    is_root = gidx == 0
    zero = jnp.zeros_like(qw)
    one = jnp.ones_like(qw)
    out_ref[0] = jnp.where(is_root, one, qw)
    out_ref[1] = jnp.where(is_root, zero, qx)
    out_ref[2] = jnp.where(is_root, zero, qy)
    out_ref[3] = jnp.where(is_root, zero, qz)
    out_ref[4] = jnp.where(is_root, zero, tx)
    out_ref[5] = jnp.where(is_root, zero, ty)
    out_ref[6] = jnp.where(is_root, zero, tz)


def _tc_build(dofs_planes, npad):
    nrow = npad // 128
    grid = nrow // 8
    return pl.pallas_call(
        _tc_build_body,
        grid=(grid,),
        in_specs=[pl.BlockSpec((4, 8, 128), lambda i: (0, i, 0))],
        out_specs=pl.BlockSpec((7, 8, 128), lambda i: (0, i, 0)),
        out_shape=jax.ShapeDtypeStruct((7, nrow, 128), jnp.float32),
    )(dofs_planes)


def _compose_group(par, own, out, g):
    ii = lax.iota(jnp.int32, L)
    r0 = ii + g * L

    def col(c):
        return lax.broadcast(jnp.int32(c), (L,))

    def loadf(ref, c):
        return plsc.bitcast(plsc.load_gather(ref, [r0, col(c)]), jnp.float32)

    pw = loadf(par, 0)
    px = loadf(par, 1)
    py = loadf(par, 2)
    pz = loadf(par, 3)
    ptx = loadf(par, 4)
    pty = loadf(par, 5)
    ptz = loadf(par, 6)
    pa = plsc.load_gather(par, [r0, col(7)])

    cw = loadf(own, 0)
    cx = loadf(own, 1)
    cy = loadf(own, 2)
    cz = loadf(own, 3)
    ctx = loadf(own, 4)
    cty = loadf(own, 5)
    ctz = loadf(own, 6)

    nw_ = pw * cw - px * cx - py * cy - pz * cz
    nx = pw * cx + px * cw + py * cz - pz * cy
    ny = pw * cy - px * cz + py * cw + pz * cx
    nz = pw * cz + px * cy - py * cx + pz * cw
    ux = py * ctz - pz * cty
    uy = pz * ctx - px * ctz
    uz = px * cty - py * ctx
    wx = py * uz - pz * uy
    wy = pz * ux - px * uz
    wz = px * uy - py * ux
    two = jnp.float32(2.0)
    ntx = ptx + ctx + two * (pw * ux + wx)
    nty = pty + cty + two * (pw * uy + wy)
    ntz = ptz + ctz + two * (pw * uz + wz)

    def storef(c, v):
        plsc.store_scatter(out, [r0, col(c)], plsc.bitcast(v, jnp.int32))

    storef(0, nw_)
    storef(1, nx)
    storef(2, ny)
    storef(3, nz)
    storef(4, ntx)
    storef(5, nty)
    storef(6, ntz)
    plsc.store_scatter(out, [r0, col(7)], pa)
    return pa


def _pd_step_body(a_hbm, b_hbm, flags_hbm, own, par, idx, flagv, shared, sem, *, wpn, npad):
    cid = lax.axis_index("c")
    sid = lax.axis_index("s")
    wid = sid * NC + cid
    base = wid * wpn

    spn = npad // NS
    pltpu.sync_copy(
        a_hbm.at[pl.ds(sid * spn, spn)], shared.at[pl.ds(sid * spn, spn)]
    )
    pltpu.sync_copy(a_hbm.at[pl.ds(base, wpn)], own)

    def ext_body(g, carry):
        ii = lax.iota(jnp.int32, L)
        r0 = ii + g * L
        c7 = lax.broadcast(jnp.int32(7), (L,))
        idx[pl.ds(g * L, L)] = plsc.load_gather(own, [r0, c7])
        return carry

    lax.fori_loop(0, wpn // L, ext_body, 0)

    plsc.subcore_barrier()

    nchunk = wpn // GCHUNK
    descs = []
    for c in range(nchunk):
        descs.append(
            pltpu.async_copy(
                shared.at[idx.at[pl.ds(c * GCHUNK, GCHUNK)]],
                par.at[pl.ds(c * GCHUNK, GCHUNK)],
                sem,
            )
        )
    for dsc in descs:
        dsc.wait()

    def comp_body(g, acc):
        pa = _compose_group(par, own, own, g)
        return acc | pa

    acc = lax.fori_loop(
        0, wpn // L, comp_body, jnp.zeros((L,), jnp.int32)
    )
    flagv[0] = acc

    pltpu.sync_copy(own, b_hbm.at[pl.ds(base, wpn)])
    pltpu.sync_copy(flagv, flags_hbm.at[pl.ds(wid, 1)])


def _pd_step(a, npad):
    wpn = npad // NW
    mesh = plsc.VectorSubcoreMesh(
        core_axis_name="c", subcore_axis_name="s", num_cores=NC, num_subcores=NS
    )
    return pl.kernel(
        functools.partial(_pd_step_body, wpn=wpn, npad=npad),
        out_type=(
            jax.ShapeDtypeStruct((npad, 8), jnp.int32),
            jax.ShapeDtypeStruct((NW, L), jnp.int32),
        ),
        mesh=mesh,
        compiler_params=pltpu.CompilerParams(
            use_tc_tiling_on_sc=False, needs_layout_passes=False
        ),
        scratch_types=[
            pltpu.VMEM((wpn, 8), jnp.int32),
            pltpu.VMEM((wpn, 8), jnp.int32),
            pltpu.VMEM((wpn,), jnp.int32),
            pltpu.VMEM((1, L), jnp.int32),
            pltpu.VMEM_SHARED((npad, 8), jnp.int32),
            pltpu.SemaphoreType.DMA,
        ],
    )(a)


def kernel(dofs, parent):
    n = dofs.shape[0]
    npad = -(-n // (NW * GCHUNK)) * (NW * GCHUNK)

    dofs4 = jnp.pad(dofs[:, :4], ((0, npad - n), (0, 0)))
    dofs_planes = dofs4.T.reshape(4, npad // 128, 128)
    planes = _tc_build(dofs_planes, npad).reshape(7, npad)

    anc = jnp.pad(parent.astype(jnp.int32), (0, npad - n))
    planes_i = lax.bitcast_convert_type(planes, jnp.int32)
    rows = jnp.concatenate([planes_i, anc[None, :]], axis=0).T

    def cond(state):
        _, cont, r = state
        return jnp.logical_and(cont, r < PD_ROUNDS)

    def body(state):
        rows_c, _, r = state
        rows_n, flags = _pd_step(rows_c, npad)
        return rows_n, jnp.any(flags != 0), r + 1

    rows = lax.while_loop(cond, body, (rows, jnp.bool_(True), 0))[0]

    return lax.bitcast_convert_type(rows[:n, 4:7], jnp.float32)


# speedup vs baseline: 38.2160x; 1.0751x over previous
import functools

import jax
import jax.numpy as jnp
from jax import lax
from jax.experimental import pallas as pl
from jax.experimental.pallas import tpu as pltpu
from jax.experimental.pallas import tpu_sc as plsc

L = 16
NC, NS = 2, 16
NW = NC * NS
GCHUNK = 128
PD_ROUNDS = 8


def _tc_build_body(dofs_ref, out_ref, *, rpb):
    phi_p = dofs_ref[0]
    theta = dofs_ref[1]
    d = dofs_ref[2]
    phi_c = dofs_ref[3]

    cp, sp = jnp.cos(0.5 * phi_p), jnp.sin(0.5 * phi_p)
    ct, st = jnp.cos(0.5 * theta), jnp.sin(0.5 * theta)
    cc, sc_ = jnp.cos(0.5 * phi_c), jnp.sin(0.5 * phi_c)
    w1 = cp * ct
    x1 = sp * ct
    y1 = -sp * st
    z1 = cp * st
    qw = w1 * cc - x1 * sc_
    qx = w1 * sc_ + x1 * cc
    qy = y1 * cc + z1 * sc_
    qz = z1 * cc - y1 * sc_
    ctf, stf = 1.0 - 2.0 * st * st, 2.0 * st * ct
    cpf, spf = 1.0 - 2.0 * sp * sp, 2.0 * sp * cp
    tx = d * ctf
    ty = cpf * d * stf
    tz = spf * d * stf

    sub = lax.broadcasted_iota(jnp.int32, (rpb, 128), 0)
    lane = lax.broadcasted_iota(jnp.int32, (rpb, 128), 1)
    gidx = (pl.program_id(0) * rpb + sub) * 128 + lane
    is_root = gidx == 0
    zero = jnp.zeros_like(qw)
    one = jnp.ones_like(qw)
    out_ref[0] = jnp.where(is_root, one, qw)
    out_ref[1] = jnp.where(is_root, zero, qx)
    out_ref[2] = jnp.where(is_root, zero, qy)
    out_ref[3] = jnp.where(is_root, zero, qz)
    out_ref[4] = jnp.where(is_root, zero, tx)
    out_ref[5] = jnp.where(is_root, zero, ty)
    out_ref[6] = jnp.where(is_root, zero, tz)


def _tc_build(dofs_planes, npad):
    nrow = npad // 128
    grid = 10
    rpb = nrow // grid
    return pl.pallas_call(
        functools.partial(_tc_build_body, rpb=rpb),
        grid=(grid,),
        in_specs=[pl.BlockSpec((4, rpb, 128), lambda i: (0, i, 0))],
        out_specs=pl.BlockSpec((7, rpb, 128), lambda i: (0, i, 0)),
        out_shape=jax.ShapeDtypeStruct((7, nrow, 128), jnp.float32),
    )(dofs_planes)


def _compose_group(par, own, out, g):
    ii = lax.iota(jnp.int32, L)
    r0 = ii + g * L

    def col(c):
        return lax.broadcast(jnp.int32(c), (L,))

    def loadf(ref, c):
        return plsc.bitcast(plsc.load_gather(ref, [r0, col(c)]), jnp.float32)

    pw = loadf(par, 0)
    px = loadf(par, 1)
    py = loadf(par, 2)
    pz = loadf(par, 3)
    ptx = loadf(par, 4)
    pty = loadf(par, 5)
    ptz = loadf(par, 6)
    pa = plsc.load_gather(par, [r0, col(7)])

    cw = loadf(own, 0)
    cx = loadf(own, 1)
    cy = loadf(own, 2)
    cz = loadf(own, 3)
    ctx = loadf(own, 4)
    cty = loadf(own, 5)
    ctz = loadf(own, 6)

    nw_ = pw * cw - px * cx - py * cy - pz * cz
    nx = pw * cx + px * cw + py * cz - pz * cy
    ny = pw * cy - px * cz + py * cw + pz * cx
    nz = pw * cz + px * cy - py * cx + pz * cw
    ux = py * ctz - pz * cty
    uy = pz * ctx - px * ctz
    uz = px * cty - py * ctx
    wx = py * uz - pz * uy
    wy = pz * ux - px * uz
    wz = px * uy - py * ux
    two = jnp.float32(2.0)
    ntx = ptx + ctx + two * (pw * ux + wx)
    nty = pty + cty + two * (pw * uy + wy)
    ntz = ptz + ctz + two * (pw * uz + wz)

    def storef(c, v):
        plsc.store_scatter(out, [r0, col(c)], plsc.bitcast(v, jnp.int32))

    storef(0, nw_)
    storef(1, nx)
    storef(2, ny)
    storef(3, nz)
    storef(4, ntx)
    storef(5, nty)
    storef(6, ntz)
    plsc.store_scatter(out, [r0, col(7)], pa)
    return pa


def _pd_step_body(a_hbm, b_hbm, flags_hbm, own, par, idx, flagv, shared, sem, *, wpn, npad):
    cid = lax.axis_index("c")
    sid = lax.axis_index("s")
    wid = sid * NC + cid
    base = wid * wpn

    spn = npad // NS
    pltpu.sync_copy(
        a_hbm.at[pl.ds(sid * spn, spn)], shared.at[pl.ds(sid * spn, spn)]
    )
    pltpu.sync_copy(a_hbm.at[pl.ds(base, wpn)], own)

    def ext_body(g, carry):
        ii = lax.iota(jnp.int32, L)
        r0 = ii + g * L
        c7 = lax.broadcast(jnp.int32(7), (L,))
        idx[pl.ds(g * L, L)] = plsc.load_gather(own, [r0, c7])
        return carry

    lax.fori_loop(0, wpn // L, ext_body, 0)

    plsc.subcore_barrier()

    nchunk = wpn // GCHUNK
    descs = []
    for c in range(nchunk):
        descs.append(
            pltpu.async_copy(
                shared.at[idx.at[pl.ds(c * GCHUNK, GCHUNK)]],
                par.at[pl.ds(c * GCHUNK, GCHUNK)],
                sem,
            )
        )
    for dsc in descs:
        dsc.wait()

    def comp_body(g, acc):
        pa = _compose_group(par, own, own, g)
        return acc | pa

    acc = lax.fori_loop(
        0, wpn // L, comp_body, jnp.zeros((L,), jnp.int32)
    )
    flagv[0] = acc

    pltpu.sync_copy(own, b_hbm.at[pl.ds(base, wpn)])
    pltpu.sync_copy(flagv, flags_hbm.at[pl.ds(wid, 1)])


def _pd_step(a, npad):
    wpn = npad // NW
    mesh = plsc.VectorSubcoreMesh(
        core_axis_name="c", subcore_axis_name="s", num_cores=NC, num_subcores=NS
    )
    return pl.kernel(
        functools.partial(_pd_step_body, wpn=wpn, npad=npad),
        out_type=(
            jax.ShapeDtypeStruct((npad, 8), jnp.int32),
            jax.ShapeDtypeStruct((NW, L), jnp.int32),
        ),
        mesh=mesh,
        compiler_params=pltpu.CompilerParams(
            use_tc_tiling_on_sc=False, needs_layout_passes=False
        ),
        scratch_types=[
            pltpu.VMEM((wpn, 8), jnp.int32),
            pltpu.VMEM((wpn, 8), jnp.int32),
            pltpu.VMEM((wpn,), jnp.int32),
            pltpu.VMEM((1, L), jnp.int32),
            pltpu.VMEM_SHARED((npad, 8), jnp.int32),
            pltpu.SemaphoreType.DMA,
        ],
    )(a)


def kernel(dofs, parent):
    n = dofs.shape[0]
    npad = -(-n // (NW * GCHUNK)) * (NW * GCHUNK)

    dofs4 = jnp.pad(dofs[:, :4], ((0, npad - n), (0, 0)))
    dofs_planes = dofs4.T.reshape(4, npad // 128, 128)
    planes = _tc_build(dofs_planes, npad).reshape(7, npad)

    anc = jnp.pad(parent.astype(jnp.int32), (0, npad - n))
    planes_i = lax.bitcast_convert_type(planes, jnp.int32)
    rows = jnp.concatenate([planes_i, anc[None, :]], axis=0).T

    def cond(state):
        _, cont, r = state
        return jnp.logical_and(cont, r < PD_ROUNDS)

    def body(state):
        rows_c, _, r = state
        rows_n, flags = _pd_step(rows_c, npad)
        return rows_n, jnp.any(flags != 0), r + 1

    rows = lax.while_loop(cond, body, (rows, jnp.bool_(True), 0))[0]

    return lax.bitcast_convert_type(rows[:n, 4:7], jnp.float32)


# speedup vs baseline: 39.2284x; 1.0265x over previous
import functools

import jax
import jax.numpy as jnp
from jax import lax
from jax.experimental import pallas as pl
from jax.experimental.pallas import tpu as pltpu
from jax.experimental.pallas import tpu_sc as plsc

L = 16
NC, NS = 2, 16
NW = NC * NS
GCHUNK = 128
PD_ROUNDS = 8


def _tc_build_body(dofs_ref, out_ref, *, rpb):
    phi_p = dofs_ref[0]
    theta = dofs_ref[1]
    d = dofs_ref[2]
    phi_c = dofs_ref[3]

    cp, sp = jnp.cos(0.5 * phi_p), jnp.sin(0.5 * phi_p)
    ct, st = jnp.cos(0.5 * theta), jnp.sin(0.5 * theta)
    cc, sc_ = jnp.cos(0.5 * phi_c), jnp.sin(0.5 * phi_c)
    w1 = cp * ct
    x1 = sp * ct
    y1 = -sp * st
    z1 = cp * st
    qw = w1 * cc - x1 * sc_
    qx = w1 * sc_ + x1 * cc
    qy = y1 * cc + z1 * sc_
    qz = z1 * cc - y1 * sc_
    ctf, stf = 1.0 - 2.0 * st * st, 2.0 * st * ct
    cpf, spf = 1.0 - 2.0 * sp * sp, 2.0 * sp * cp
    tx = d * ctf
    ty = cpf * d * stf
    tz = spf * d * stf

    sub = lax.broadcasted_iota(jnp.int32, (rpb, 128), 0)
    lane = lax.broadcasted_iota(jnp.int32, (rpb, 128), 1)
    gidx = (pl.program_id(0) * rpb + sub) * 128 + lane
    is_root = gidx == 0
    zero = jnp.zeros_like(qw)
    one = jnp.ones_like(qw)
    out_ref[0] = jnp.where(is_root, one, qw)
    out_ref[1] = jnp.where(is_root, zero, qx)
    out_ref[2] = jnp.where(is_root, zero, qy)
    out_ref[3] = jnp.where(is_root, zero, qz)
    out_ref[4] = jnp.where(is_root, zero, tx)
    out_ref[5] = jnp.where(is_root, zero, ty)
    out_ref[6] = jnp.where(is_root, zero, tz)


def _tc_build(dofs_planes, npad):
    nrow = npad // 128
    grid = 2
    rpb = nrow // grid
    return pl.pallas_call(
        functools.partial(_tc_build_body, rpb=rpb),
        grid=(grid,),
        in_specs=[pl.BlockSpec((4, rpb, 128), lambda i: (0, i, 0))],
        out_specs=pl.BlockSpec((7, rpb, 128), lambda i: (0, i, 0)),
        out_shape=jax.ShapeDtypeStruct((7, nrow, 128), jnp.float32),
    )(dofs_planes)


def _compose_group(par, own, out, g):
    ii = lax.iota(jnp.int32, L)
    r0 = ii + g * L

    def col(c):
        return lax.broadcast(jnp.int32(c), (L,))

    def loadf(ref, c):
        return plsc.bitcast(plsc.load_gather(ref, [r0, col(c)]), jnp.float32)

    pw = loadf(par, 0)
    px = loadf(par, 1)
    py = loadf(par, 2)
    pz = loadf(par, 3)
    ptx = loadf(par, 4)
    pty = loadf(par, 5)
    ptz = loadf(par, 6)
    pa = plsc.load_gather(par, [r0, col(7)])

    cw = loadf(own, 0)
    cx = loadf(own, 1)
    cy = loadf(own, 2)
    cz = loadf(own, 3)
    ctx = loadf(own, 4)
    cty = loadf(own, 5)
    ctz = loadf(own, 6)

    nw_ = pw * cw - px * cx - py * cy - pz * cz
    nx = pw * cx + px * cw + py * cz - pz * cy
    ny = pw * cy - px * cz + py * cw + pz * cx
    nz = pw * cz + px * cy - py * cx + pz * cw
    ux = py * ctz - pz * cty
    uy = pz * ctx - px * ctz
    uz = px * cty - py * ctx
    wx = py * uz - pz * uy
    wy = pz * ux - px * uz
    wz = px * uy - py * ux
    two = jnp.float32(2.0)
    ntx = ptx + ctx + two * (pw * ux + wx)
    nty = pty + cty + two * (pw * uy + wy)
    ntz = ptz + ctz + two * (pw * uz + wz)

    def storef(c, v):
        plsc.store_scatter(out, [r0, col(c)], plsc.bitcast(v, jnp.int32))

    storef(0, nw_)
    storef(1, nx)
    storef(2, ny)
    storef(3, nz)
    storef(4, ntx)
    storef(5, nty)
    storef(6, ntz)
    plsc.store_scatter(out, [r0, col(7)], pa)
    return pa


def _pd_step_body(a_hbm, b_hbm, flags_hbm, own, par, idx, flagv, shared, sem, sem2, *, wpn, npad):
    cid = lax.axis_index("c")
    sid = lax.axis_index("s")
    wid = sid * NC + cid
    base = wid * wpn

    spn = npad // NS
    stg = pltpu.async_copy(
        a_hbm.at[pl.ds(sid * spn, spn)], shared.at[pl.ds(sid * spn, spn)], sem2
    )
    pltpu.sync_copy(a_hbm.at[pl.ds(base, wpn)], own)

    def ext_body(g, carry):
        ii = lax.iota(jnp.int32, L)
        r0 = ii + g * L
        c7 = lax.broadcast(jnp.int32(7), (L,))
        idx[pl.ds(g * L, L)] = plsc.load_gather(own, [r0, c7])
        return carry

    lax.fori_loop(0, wpn // L, ext_body, 0)

    stg.wait()
    plsc.subcore_barrier()

    nchunk = wpn // GCHUNK
    descs = []
    for c in range(nchunk):
        descs.append(
            pltpu.async_copy(
                shared.at[idx.at[pl.ds(c * GCHUNK, GCHUNK)]],
                par.at[pl.ds(c * GCHUNK, GCHUNK)],
                sem,
            )
        )
    for dsc in descs:
        dsc.wait()

    def comp_body(g, acc):
        pa = _compose_group(par, own, own, g)
        return acc | pa

    acc = lax.fori_loop(
        0, wpn // L, comp_body, jnp.zeros((L,), jnp.int32)
    )
    flagv[0] = acc

    pltpu.sync_copy(own, b_hbm.at[pl.ds(base, wpn)])
    pltpu.sync_copy(flagv, flags_hbm.at[pl.ds(wid, 1)])


def _pd_step(a, npad):
    wpn = npad // NW
    mesh = plsc.VectorSubcoreMesh(
        core_axis_name="c", subcore_axis_name="s", num_cores=NC, num_subcores=NS
    )
    return pl.kernel(
        functools.partial(_pd_step_body, wpn=wpn, npad=npad),
        out_type=(
            jax.ShapeDtypeStruct((npad, 8), jnp.int32),
            jax.ShapeDtypeStruct((NW, L), jnp.int32),
        ),
        mesh=mesh,
        compiler_params=pltpu.CompilerParams(
            use_tc_tiling_on_sc=False, needs_layout_passes=False
        ),
        scratch_types=[
            pltpu.VMEM((wpn, 8), jnp.int32),
            pltpu.VMEM((wpn, 8), jnp.int32),
            pltpu.VMEM((wpn,), jnp.int32),
            pltpu.VMEM((1, L), jnp.int32),
            pltpu.VMEM_SHARED((npad, 8), jnp.int32),
            pltpu.SemaphoreType.DMA,
            pltpu.SemaphoreType.DMA,
        ],
    )(a)


def kernel(dofs, parent):
    n = dofs.shape[0]
    npad = -(-n // (NW * GCHUNK)) * (NW * GCHUNK)

    dofs4 = jnp.pad(dofs[:, :4], ((0, npad - n), (0, 0)))
    dofs_planes = dofs4.T.reshape(4, npad // 128, 128)
    planes = _tc_build(dofs_planes, npad).reshape(7, npad)

    anc = jnp.pad(parent.astype(jnp.int32), (0, npad - n))
    planes_i = lax.bitcast_convert_type(planes, jnp.int32)
    rows = jnp.concatenate([planes_i, anc[None, :]], axis=0).T

    def cond(state):
        _, cont, r = state
        return jnp.logical_and(cont, r < PD_ROUNDS)

    def body(state):
        rows_c, _, r = state
        rows_n, flags = _pd_step(rows_c, npad)
        return rows_n, jnp.any(flags != 0), r + 1

    rows = lax.while_loop(cond, body, (rows, jnp.bool_(True), 0))[0]

    return lax.bitcast_convert_type(rows[:n, 4:7], jnp.float32)
